# baseline (speedup 1.0000x reference)
import functools

import jax
import jax.numpy as jnp
from jax import lax
from jax.experimental import pallas as pl
from jax.experimental.pallas import tpu as pltpu
from jax.experimental.pallas import tpu_sc as plsc

_K = 16
_SIGMA = 0.0001
_BIG = 2**30

_noise_cache = {}


def _scaled_noise(b, n, m, dtype):
    ck = (b, n, m, jnp.dtype(dtype).name)
    if ck not in _noise_cache:
        with jax.default_device(jax.devices("cpu")[0]):
            nk = jax.random.fold_in(jax.random.key(0), 1)
            noise = jax.random.normal(nk, (b, n, 1, m), dtype=dtype)
            _noise_cache[ck] = jax.block_until_ready(
                (noise * _SIGMA).reshape(b * n, m))
    return _noise_cache[ck]


def _xlane_min(v, lanes):
    for s in (1, 2, 4, 8):
        v = jnp.minimum(v, v.at[lanes ^ s].get(mode="promise_in_bounds"))
    return v


def _sc_topk_body(rw, m, x_hbm, noise_hbm, idx_hbm, xv, nv, ist):
    nc = 2
    wid = lax.axis_index("s") * nc + lax.axis_index("c")
    base = wid * rw
    pltpu.sync_copy(x_hbm.at[wid], xv)
    pltpu.sync_copy(noise_hbm.at[wid], nv)
    lanes = lax.iota(jnp.int32, 16)
    inf16 = jnp.full((16,), jnp.inf, jnp.float32)
    zero16 = jnp.zeros((16,), jnp.int32)

    def row_body(r, _):
        rsplat = jnp.full((16,), r, jnp.int32)

        def acc_body(j, mj):
            a1, aj1, a2, aj2, b1, bj1, b2, bj2 = mj
            ja = 2 * j
            sla = pl.ds(ja * 16, 16)
            xa = xv[r, sla] + nv[r, sla]
            jspa = jnp.full((16,), ja, jnp.int32)
            c1 = xa < a1
            c2 = xa < a2
            a2n = jnp.where(c1, a1, jnp.where(c2, xa, a2))
            aj2n = jnp.where(c1, aj1, jnp.where(c2, jspa, aj2))
            a1n = jnp.where(c1, xa, a1)
            aj1n = jnp.where(c1, jspa, aj1)
            jb = 2 * j + 1
            slb = pl.ds(jb * 16, 16)
            xb = xv[r, slb] + nv[r, slb]
            jspb = jnp.full((16,), jb, jnp.int32)
            d1 = xb < b1
            d2 = xb < b2
            b2n = jnp.where(d1, b1, jnp.where(d2, xb, b2))
            bj2n = jnp.where(d1, bj1, jnp.where(d2, jspb, bj2))
            b1n = jnp.where(d1, xb, b1)
            bj1n = jnp.where(d1, jspb, bj1)
            return (a1n, aj1n, a2n, aj2n, b1n, bj1n, b2n, bj2n)

        a1, aj1, a2, aj2, b1, bj1, b2, bj2 = lax.fori_loop(
            0, m // 32, acc_body,
            (inf16, zero16, inf16, zero16, inf16, zero16, inf16, zero16),
            unroll=4)
        aw = (a1 < b1) | ((a1 == b1) & (aj1 < bj1))
        m1 = jnp.where(aw, a1, b1)
        j1 = jnp.where(aw, aj1, bj1)
        sx = jnp.where(aw, a2, b2)
        sxj = jnp.where(aw, aj2, bj2)
        sy = jnp.where(aw, b1, a1)
        syj = jnp.where(aw, bj1, aj1)
        sw = (sx < sy) | ((sx == sy) & (sxj < syj))
        m2 = jnp.where(sw, sx, sy)
        j2 = jnp.where(sw, sxj, syj)

        def ext_body(k, mj):
            m1, j1, m2, j2, ivec = mj
            mv = _xlane_min(m1, lanes)
            gi = _xlane_min(jnp.where(m1 == mv, j1 * 16 + lanes, _BIG), lanes)
            ivec = jnp.where(lanes == k, gi, ivec)
            plsc.store_scatter(xv, [rsplat, gi], inf16, mask=lanes == 0)
            lm = lanes == (gi & 15)
            m1 = jnp.where(lm, m2, m1)
            j1 = jnp.where(lm, j2, j1)
            m2 = jnp.where(lm, inf16, m2)

            def rescan(args):
                m1, j1 = args

                def rescan_body(c, mj2):
                    mr, jr = mj2
                    jvec = c * 16 + lanes
                    vals = (plsc.load_gather(xv, [rsplat, jvec * 16 + (gi & 15)])
                            + plsc.load_gather(nv, [rsplat, jvec * 16 + (gi & 15)]))
                    upd = vals < mr
                    return (jnp.where(upd, vals, mr), jnp.where(upd, jvec, jr))

                mr, jr = lax.fori_loop(0, m // 256, rescan_body,
                                       (inf16, zero16), unroll=4)
                mbest = _xlane_min(mr, lanes)
                jbest = _xlane_min(jnp.where(mr == mbest, jr, _BIG), lanes)
                return (jnp.where(lm, mbest, m1), jnp.where(lm, jbest, j1))

            need = jnp.any(lm & (m1 == inf16))
            m1, j1 = lax.cond(need, rescan, lambda a: a, (m1, j1))
            return (m1, j1, m2, j2, ivec)

        _, _, _, _, ivec = lax.fori_loop(
            0, _K, ext_body, (m1, j1, m2, j2, zero16))
        ist[r, :] = ivec
        return 0

    lax.fori_loop(0, rw, row_body, 0)
    pltpu.sync_copy(ist, idx_hbm.at[pl.ds(base, rw)])


def _onehot_kernel(idx_ref, out_ref):
    iota = jax.lax.broadcasted_iota(jnp.int32, out_ref.shape, 2)
    out_ref[...] = (iota == idx_ref[...][:, :, None]).astype(jnp.float32)


def kernel(x):
    b, n, m = x.shape
    rows = b * n
    noise = _scaled_noise(b, n, m, x.dtype).reshape(b, n, m)

    rw = rows // 32
    assert rw == n and b == 32, "SC row mapping assumes b == 32 subcores"
    mesh = plsc.VectorSubcoreMesh(core_axis_name="c", subcore_axis_name="s")
    sc_topk = functools.partial(
        pl.kernel,
        out_type=jax.ShapeDtypeStruct((rows, _K), jnp.int32),
        mesh=mesh,
        scratch_types=[
            pltpu.VMEM((rw, m), jnp.float32),
            pltpu.VMEM((rw, m), jnp.float32),
            pltpu.VMEM((rw, _K), jnp.int32),
        ],
        compiler_params=pltpu.CompilerParams(use_tc_tiling_on_sc=False, needs_layout_passes=False),
    )(functools.partial(_sc_topk_body, rw, m))
    idx = sc_topk(x, noise)

    r2 = 32 if rows % 32 == 0 else 1
    out = pl.pallas_call(
        _onehot_kernel,
        grid=(rows // r2,),
        in_specs=[pl.BlockSpec((r2, _K), lambda i: (i, 0))],
        out_specs=pl.BlockSpec((r2, _K, m), lambda i: (i, 0, 0)),
        out_shape=jax.ShapeDtypeStruct((rows, _K, m), jnp.float32),
    )(idx)
    return out.reshape(b, n, _K, m)


# speedup vs baseline: 1.3242x; 1.3242x over previous
import functools

import jax
import jax.numpy as jnp
from jax import lax
from jax.experimental import pallas as pl
from jax.experimental.pallas import tpu as pltpu
from jax.experimental.pallas import tpu_sc as plsc

_K = 16
_SIGMA = 0.0001
_BIG = 2**30

_noise_cache = {}


def _scaled_noise(b, n, m, dtype):
    ck = (b, n, m, jnp.dtype(dtype).name)
    if ck not in _noise_cache:
        with jax.ensure_compile_time_eval():
            with jax.default_device(jax.devices("cpu")[0]):
                nk = jax.random.fold_in(jax.random.key(0), 1)
                noise = jax.random.normal(nk, (b, n, 1, m), dtype=dtype)
                _noise_cache[ck] = jax.block_until_ready(
                    (noise * _SIGMA).reshape(b * n, m))
    return _noise_cache[ck]


def _xlane_min(v, lanes):
    for s in (1, 2, 4, 8):
        v = jnp.minimum(v, v.at[lanes ^ s].get(mode="promise_in_bounds"))
    return v


def _sc_topk_body(rw, m, x_hbm, noise_hbm, idx_hbm, xv, nv, ist):
    nc = 2
    wid = lax.axis_index("s") * nc + lax.axis_index("c")
    base = wid * rw
    pltpu.sync_copy(x_hbm.at[wid], xv)
    pltpu.sync_copy(noise_hbm.at[wid], nv)
    lanes = lax.iota(jnp.int32, 16)
    inf16 = jnp.full((16,), jnp.inf, jnp.float32)
    zero16 = jnp.zeros((16,), jnp.int32)

    def row_body(r, _):
        rsplat = jnp.full((16,), r, jnp.int32)

        def acc_body(j, mj):
            a1, aj1, a2, aj2, b1, bj1, b2, bj2 = mj
            ja = 2 * j
            sla = pl.ds(ja * 16, 16)
            xa = xv[r, sla] + nv[r, sla]
            jspa = jnp.full((16,), ja, jnp.int32)
            c1 = xa < a1
            c2 = xa < a2
            a2n = jnp.where(c1, a1, jnp.where(c2, xa, a2))
            aj2n = jnp.where(c1, aj1, jnp.where(c2, jspa, aj2))
            a1n = jnp.where(c1, xa, a1)
            aj1n = jnp.where(c1, jspa, aj1)
            jb = 2 * j + 1
            slb = pl.ds(jb * 16, 16)
            xb = xv[r, slb] + nv[r, slb]
            jspb = jnp.full((16,), jb, jnp.int32)
            d1 = xb < b1
            d2 = xb < b2
            b2n = jnp.where(d1, b1, jnp.where(d2, xb, b2))
            bj2n = jnp.where(d1, bj1, jnp.where(d2, jspb, bj2))
            b1n = jnp.where(d1, xb, b1)
            bj1n = jnp.where(d1, jspb, bj1)
            return (a1n, aj1n, a2n, aj2n, b1n, bj1n, b2n, bj2n)

        a1, aj1, a2, aj2, b1, bj1, b2, bj2 = lax.fori_loop(
            0, m // 32, acc_body,
            (inf16, zero16, inf16, zero16, inf16, zero16, inf16, zero16),
            unroll=4)
        aw = (a1 < b1) | ((a1 == b1) & (aj1 < bj1))
        m1 = jnp.where(aw, a1, b1)
        j1 = jnp.where(aw, aj1, bj1)
        sx = jnp.where(aw, a2, b2)
        sxj = jnp.where(aw, aj2, bj2)
        sy = jnp.where(aw, b1, a1)
        syj = jnp.where(aw, bj1, aj1)
        sw = (sx < sy) | ((sx == sy) & (sxj < syj))
        m2 = jnp.where(sw, sx, sy)
        j2 = jnp.where(sw, sxj, syj)

        def ext_body(k, mj):
            m1, j1, m2, j2, ivec = mj
            mv = _xlane_min(m1, lanes)
            gi = _xlane_min(jnp.where(m1 == mv, j1 * 16 + lanes, _BIG), lanes)
            ivec = jnp.where(lanes == k, gi, ivec)
            plsc.store_scatter(xv, [rsplat, gi], inf16, mask=lanes == 0)
            lm = lanes == (gi & 15)
            m1 = jnp.where(lm, m2, m1)
            j1 = jnp.where(lm, j2, j1)
            m2 = jnp.where(lm, inf16, m2)

            def rescan(args):
                m1, j1 = args

                def rescan_body(c, mj2):
                    mr, jr = mj2
                    jvec = c * 16 + lanes
                    vals = (plsc.load_gather(xv, [rsplat, jvec * 16 + (gi & 15)])
                            + plsc.load_gather(nv, [rsplat, jvec * 16 + (gi & 15)]))
                    upd = vals < mr
                    return (jnp.where(upd, vals, mr), jnp.where(upd, jvec, jr))

                mr, jr = lax.fori_loop(0, m // 256, rescan_body,
                                       (inf16, zero16), unroll=4)
                mbest = _xlane_min(mr, lanes)
                jbest = _xlane_min(jnp.where(mr == mbest, jr, _BIG), lanes)
                return (jnp.where(lm, mbest, m1), jnp.where(lm, jbest, j1))

            need = jnp.any(lm & (m1 == inf16))
            m1, j1 = lax.cond(need, rescan, lambda a: a, (m1, j1))
            return (m1, j1, m2, j2, ivec)

        _, _, _, _, ivec = lax.fori_loop(
            0, _K, ext_body, (m1, j1, m2, j2, zero16))
        ist[r, :] = ivec
        return 0

    lax.fori_loop(0, rw, row_body, 0)
    pltpu.sync_copy(ist, idx_hbm.at[pl.ds(base, rw)])


def _onehot_kernel(idx_ref, out_ref):
    iota = jax.lax.broadcasted_iota(jnp.int32, out_ref.shape, 2)
    out_ref[...] = (iota == idx_ref[...][:, :, None]).astype(jnp.float32)


def kernel(x):
    b, n, m = x.shape
    rows = b * n
    noise = _scaled_noise(b, n, m, x.dtype).reshape(b, n, m)

    rw = rows // 32
    assert rw == n and b == 32, "SC row mapping assumes b == 32 subcores"
    mesh = plsc.VectorSubcoreMesh(core_axis_name="c", subcore_axis_name="s")
    sc_topk = functools.partial(
        pl.kernel,
        out_type=jax.ShapeDtypeStruct((rows, _K), jnp.int32),
        mesh=mesh,
        scratch_types=[
            pltpu.VMEM((rw, m), jnp.float32),
            pltpu.VMEM((rw, m), jnp.float32),
            pltpu.VMEM((rw, _K), jnp.int32),
        ],
        compiler_params=pltpu.CompilerParams(use_tc_tiling_on_sc=False, needs_layout_passes=False),
    )(functools.partial(_sc_topk_body, rw, m))
    idx = sc_topk(x, noise)

    r2 = 32 if rows % 32 == 0 else 1
    out = pl.pallas_call(
        _onehot_kernel,
        grid=(rows // r2,),
        in_specs=[pl.BlockSpec((r2, _K), lambda i: (i, 0))],
        out_specs=pl.BlockSpec((r2, _K, m), lambda i: (i, 0, 0)),
        out_shape=jax.ShapeDtypeStruct((rows, _K, m), jnp.float32),
    )(idx)
    return out.reshape(b, n, _K, m)


# speedup vs baseline: 1.3246x; 1.0003x over previous
import functools

import jax
import jax.numpy as jnp
from jax import lax
from jax.experimental import pallas as pl
from jax.experimental.pallas import tpu as pltpu
from jax.experimental.pallas import tpu_sc as plsc

_K = 16
_SIGMA = 0.0001
_BIG = 2**30

_noise_cache = {}


def _scaled_noise(b, n, m, dtype):
    ck = (b, n, m, jnp.dtype(dtype).name)
    if ck not in _noise_cache:
        with jax.ensure_compile_time_eval():
            with jax.default_device(jax.devices("cpu")[0]):
                nk = jax.random.fold_in(jax.random.key(0), 1)
                noise = jax.random.normal(nk, (b, n, 1, m), dtype=dtype)
                _noise_cache[ck] = jax.block_until_ready(
                    (noise * _SIGMA).reshape(b * n, m))
    return _noise_cache[ck]


def _xlane_min(v, lanes):
    for s in (1, 2, 4, 8):
        v = jnp.minimum(v, v.at[lanes ^ s].get(mode="promise_in_bounds"))
    return v


def _sc_topk_body(rw, m, x_hbm, noise_hbm, idx_hbm, xv, nv, ist):
    nc = 2
    wid = lax.axis_index("s") * nc + lax.axis_index("c")
    base = wid * rw
    pltpu.sync_copy(x_hbm.at[wid], xv)
    pltpu.sync_copy(noise_hbm.at[wid], nv)
    lanes = lax.iota(jnp.int32, 16)
    inf16 = jnp.full((16,), jnp.inf, jnp.float32)
    zero16 = jnp.zeros((16,), jnp.int32)

    def row_body(r, _):
        rsplat = jnp.full((16,), r, jnp.int32)

        def acc_body(j, mj):
            a1, aj1, a2, aj2, b1, bj1, b2, bj2 = mj
            ja = 2 * j
            sla = pl.ds(ja * 16, 16)
            xa = xv[r, sla] + nv[r, sla]
            jspa = jnp.full((16,), ja, jnp.int32)
            c1 = xa < a1
            c2 = xa < a2
            a2n = jnp.where(c1, a1, jnp.where(c2, xa, a2))
            aj2n = jnp.where(c1, aj1, jnp.where(c2, jspa, aj2))
            a1n = jnp.where(c1, xa, a1)
            aj1n = jnp.where(c1, jspa, aj1)
            jb = 2 * j + 1
            slb = pl.ds(jb * 16, 16)
            xb = xv[r, slb] + nv[r, slb]
            jspb = jnp.full((16,), jb, jnp.int32)
            d1 = xb < b1
            d2 = xb < b2
            b2n = jnp.where(d1, b1, jnp.where(d2, xb, b2))
            bj2n = jnp.where(d1, bj1, jnp.where(d2, jspb, bj2))
            b1n = jnp.where(d1, xb, b1)
            bj1n = jnp.where(d1, jspb, bj1)
            return (a1n, aj1n, a2n, aj2n, b1n, bj1n, b2n, bj2n)

        a1, aj1, a2, aj2, b1, bj1, b2, bj2 = lax.fori_loop(
            0, m // 32, acc_body,
            (inf16, zero16, inf16, zero16, inf16, zero16, inf16, zero16),
            unroll=8)
        aw = (a1 < b1) | ((a1 == b1) & (aj1 < bj1))
        m1 = jnp.where(aw, a1, b1)
        j1 = jnp.where(aw, aj1, bj1)
        sx = jnp.where(aw, a2, b2)
        sxj = jnp.where(aw, aj2, bj2)
        sy = jnp.where(aw, b1, a1)
        syj = jnp.where(aw, bj1, aj1)
        sw = (sx < sy) | ((sx == sy) & (sxj < syj))
        m2 = jnp.where(sw, sx, sy)
        j2 = jnp.where(sw, sxj, syj)

        def ext_body(k, mj):
            m1, j1, m2, j2, ivec = mj
            mv = _xlane_min(m1, lanes)
            gi = _xlane_min(jnp.where(m1 == mv, j1 * 16 + lanes, _BIG), lanes)
            ivec = jnp.where(lanes == k, gi, ivec)
            plsc.store_scatter(xv, [rsplat, gi], inf16, mask=lanes == 0)
            lm = lanes == (gi & 15)
            m1 = jnp.where(lm, m2, m1)
            j1 = jnp.where(lm, j2, j1)
            m2 = jnp.where(lm, inf16, m2)

            def rescan(args):
                m1, j1 = args

                def rescan_body(c, mj2):
                    mr, jr = mj2
                    jvec = c * 16 + lanes
                    vals = (plsc.load_gather(xv, [rsplat, jvec * 16 + (gi & 15)])
                            + plsc.load_gather(nv, [rsplat, jvec * 16 + (gi & 15)]))
                    upd = vals < mr
                    return (jnp.where(upd, vals, mr), jnp.where(upd, jvec, jr))

                mr, jr = lax.fori_loop(0, m // 256, rescan_body,
                                       (inf16, zero16), unroll=4)
                mbest = _xlane_min(mr, lanes)
                jbest = _xlane_min(jnp.where(mr == mbest, jr, _BIG), lanes)
                return (jnp.where(lm, mbest, m1), jnp.where(lm, jbest, j1))

            need = jnp.any(lm & (m1 == inf16))
            m1, j1 = lax.cond(need, rescan, lambda a: a, (m1, j1))
            return (m1, j1, m2, j2, ivec)

        _, _, _, _, ivec = lax.fori_loop(
            0, _K, ext_body, (m1, j1, m2, j2, zero16))
        ist[r, :] = ivec
        return 0

    lax.fori_loop(0, rw, row_body, 0)
    pltpu.sync_copy(ist, idx_hbm.at[wid])


def _onehot_kernel(idx_ref, out_ref):
    iota = jax.lax.broadcasted_iota(jnp.int32, out_ref.shape, 3)
    out_ref[...] = (iota == idx_ref[...][:, :, :, None]).astype(jnp.float32)


def kernel(x):
    b, n, m = x.shape
    rows = b * n
    noise = _scaled_noise(b, n, m, x.dtype).reshape(b, n, m)

    rw = rows // 32
    assert rw == n and b == 32, "SC row mapping assumes b == 32 subcores"
    mesh = plsc.VectorSubcoreMesh(core_axis_name="c", subcore_axis_name="s")
    sc_topk = functools.partial(
        pl.kernel,
        out_type=jax.ShapeDtypeStruct((b, n, _K), jnp.int32),
        mesh=mesh,
        scratch_types=[
            pltpu.VMEM((rw, m), jnp.float32),
            pltpu.VMEM((rw, m), jnp.float32),
            pltpu.VMEM((rw, _K), jnp.int32),
        ],
        compiler_params=pltpu.CompilerParams(use_tc_tiling_on_sc=False, needs_layout_passes=False),
    )(functools.partial(_sc_topk_body, rw, m))
    idx = sc_topk(x, noise)

    bb = 4 if b % 4 == 0 else 1
    out = pl.pallas_call(
        _onehot_kernel,
        grid=(b // bb,),
        in_specs=[pl.BlockSpec((bb, n, _K), lambda i: (i, 0, 0))],
        out_specs=pl.BlockSpec((bb, n, _K, m), lambda i: (i, 0, 0, 0)),
        out_shape=jax.ShapeDtypeStruct((b, n, _K, m), jnp.float32),
    )(idx)
    return out


# speedup vs baseline: 2.1120x; 1.5945x over previous
import jax
import jax.numpy as jnp
from jax.experimental import pallas as pl

_K = 16
_SIGMA = 0.0001

_noise_cache = {}


def _scaled_noise(b, n, m, dtype):
    ck = (b, n, m, jnp.dtype(dtype).name)
    if ck not in _noise_cache:
        with jax.ensure_compile_time_eval():
            with jax.default_device(jax.devices("cpu")[0]):
                nk = jax.random.fold_in(jax.random.key(0), 1)
                noise = jax.random.normal(nk, (b, n, 1, m), dtype=dtype)
                _noise_cache[ck] = jax.block_until_ready(
                    (noise * _SIGMA).reshape(b * n, m))
    return _noise_cache[ck]


def _topk_idx_kernel(x_ref, noise_ref, idx_ref):
    v = x_ref[...] + noise_ref[...]
    m = v.shape[1]
    iota = jax.lax.broadcasted_iota(jnp.int32, v.shape, 1)
    cols = []
    for _ in range(_K):
        minv = jnp.min(v, axis=1, keepdims=True)
        idx = jnp.min(jnp.where(v == minv, iota, m), axis=1, keepdims=True)
        cols.append(idx)
        v = jnp.where(iota == idx, jnp.inf, v)
    idx_ref[...] = jnp.concatenate(cols, axis=1)


def _onehot_kernel(idx_ref, out_ref):
    iota = jax.lax.broadcasted_iota(jnp.int32, out_ref.shape, 2)
    out_ref[...] = (iota == idx_ref[...][:, :, None]).astype(jnp.float32)


def kernel(x):
    b, n, m = x.shape
    rows = b * n
    x2 = x.reshape(rows, m)
    noise = _scaled_noise(b, n, m, x.dtype)

    r1 = 128 if rows % 128 == 0 else 1
    idx = pl.pallas_call(
        _topk_idx_kernel,
        grid=(rows // r1,),
        in_specs=[
            pl.BlockSpec((r1, m), lambda i: (i, 0)),
            pl.BlockSpec((r1, m), lambda i: (i, 0)),
        ],
        out_specs=pl.BlockSpec((r1, _K), lambda i: (i, 0)),
        out_shape=jax.ShapeDtypeStruct((rows, _K), jnp.int32),
    )(x2, noise)

    r2 = 32 if rows % 32 == 0 else 1
    out = pl.pallas_call(
        _onehot_kernel,
        grid=(rows // r2,),
        in_specs=[pl.BlockSpec((r2, _K), lambda i: (i, 0))],
        out_specs=pl.BlockSpec((r2, _K, m), lambda i: (i, 0, 0)),
        out_shape=jax.ShapeDtypeStruct((rows, _K, m), jnp.float32),
    )(idx)
    return out.reshape(b, n, _K, m)
